# baseline (speedup 1.0000x reference)
import functools

import jax
import jax.numpy as jnp
from jax import lax
from jax.experimental import pallas as pl
from jax.experimental.pallas import tpu as pltpu
from jax.experimental.pallas import tpu_sc as plsc

_NC = 2
_NS = 16
_NW = _NC * _NS
_L = 16
_D = 64
_CHUNK = 128
_ROW_UNROLL = 4


def _rsqrt16(x):
    i = plsc.bitcast(x, jnp.int32)
    y = plsc.bitcast(jnp.int32(0x5F3759DF) - (i >> 1), jnp.float32)
    for _ in range(2):
        y = y * (1.5 - 0.5 * x * y * y)
    return y


@functools.lru_cache(maxsize=None)
def _make_kernel(n_chunks):
    mesh = plsc.VectorSubcoreMesh(core_axis_name="c", subcore_axis_name="s")

    @functools.partial(
        pl.kernel,
        mesh=mesh,
        compiler_params=pltpu.CompilerParams(
            needs_layout_passes=False, use_tc_tiling_on_sc=False
        ),
        out_type=jax.ShapeDtypeStruct((_NW, n_chunks, _CHUNK, _D), jnp.float32),
        scratch_types=[
            pltpu.VMEM((n_chunks, _CHUNK), jnp.int32),
            pltpu.VMEM((_CHUNK, _D), jnp.float32),
            pltpu.SemaphoreType.DMA,
        ],
    )
    def k(idx_hbm, tab_hbm, out_hbm, idx_v, buf, gsem):
        cid = lax.axis_index("c")
        sid = lax.axis_index("s")
        wid = sid * _NC + cid
        pltpu.sync_copy(idx_hbm.at[wid], idx_v)
        lane = lax.iota(jnp.int32, _L)
        perms = [lane ^ sh for sh in (8, 4, 2, 1)]

        def scale_row(row):
            qs = [buf[row, pl.ds(q * _L, _L)] for q in range(4)]
            s = qs[0] * qs[0] + qs[1] * qs[1] + qs[2] * qs[2] + qs[3] * qs[3]
            for p in perms:
                s = s + jnp.take_along_axis(s, p, axis=0)
            scale = jnp.where(s > 1.0, _rsqrt16(s), 1.0)
            for q in range(4):
                buf[row, pl.ds(q * _L, _L)] = qs[q] * scale

        def chunk_body(g, carry):
            pltpu.async_copy(tab_hbm.at[idx_v.at[g]], buf, gsem).wait()

            def row_body(r, carry2):
                for u in range(_ROW_UNROLL):
                    scale_row(r * _ROW_UNROLL + u)
                return carry2

            lax.fori_loop(0, _CHUNK // _ROW_UNROLL, row_body, 0)
            pltpu.sync_copy(buf, out_hbm.at[wid, g])
            return carry

        lax.fori_loop(0, n_chunks, chunk_body, 0)

    return k


def kernel(indices, node_emb):
    bsz, fields = indices.shape
    n_rows = bsz * fields
    assert n_rows % (_NW * _CHUNK) == 0, n_rows
    n_chunks = n_rows // (_NW * _CHUNK)
    idx3 = indices.reshape(_NW, n_chunks, _CHUNK)
    out = _make_kernel(n_chunks)(idx3, node_emb)
    return out.reshape(bsz, fields, node_emb.shape[1])


# speedup vs baseline: 1.1452x; 1.1452x over previous
import functools

import jax
import jax.numpy as jnp
from jax import lax
from jax.experimental import pallas as pl
from jax.experimental.pallas import tpu as pltpu
from jax.experimental.pallas import tpu_sc as plsc

_NC = 2
_NS = 16
_NW = _NC * _NS
_L = 16
_D = 64
_CHUNK = 128
_NBUF = 4
_ROW_UNROLL = 4


def _rsqrt16(x):
    i = plsc.bitcast(x, jnp.int32)
    y = plsc.bitcast(jnp.int32(0x5F3759DF) - (i >> 1), jnp.float32)
    for _ in range(2):
        y = y * (1.5 - 0.5 * x * y * y)
    return y


@functools.lru_cache(maxsize=None)
def _make_kernel(n_chunks):
    assert n_chunks % _NBUF == 0 and n_chunks >= 2 * _NBUF
    mesh = plsc.VectorSubcoreMesh(core_axis_name="c", subcore_axis_name="s")

    @functools.partial(
        pl.kernel,
        mesh=mesh,
        compiler_params=pltpu.CompilerParams(
            needs_layout_passes=False, use_tc_tiling_on_sc=False
        ),
        out_type=jax.ShapeDtypeStruct((_NW, n_chunks, _CHUNK, _D), jnp.float32),
        scratch_types=[
            pltpu.VMEM((n_chunks, _CHUNK), jnp.int32),
            *([pltpu.VMEM((_CHUNK, _D), jnp.float32)] * _NBUF),
            *([pltpu.SemaphoreType.DMA] * (2 * _NBUF)),
        ],
    )
    def k(idx_hbm, tab_hbm, out_hbm, idx_v, *bufs_sems):
        bufs = bufs_sems[:_NBUF]
        gsems = bufs_sems[_NBUF : 2 * _NBUF]
        osems = bufs_sems[2 * _NBUF :]
        cid = lax.axis_index("c")
        sid = lax.axis_index("s")
        wid = sid * _NC + cid
        pltpu.sync_copy(idx_hbm.at[wid], idx_v)
        lane = lax.iota(jnp.int32, _L)
        perms = [lane ^ sh for sh in (8, 4, 2, 1)]

        def start_gather(g, j):
            pltpu.async_copy(tab_hbm.at[idx_v.at[g]], bufs[j], gsems[j])

        def wait_gather(j):
            pltpu.make_async_copy(tab_hbm.at[idx_v.at[0]], bufs[j], gsems[j]).wait()

        def start_out(h, j):
            pltpu.async_copy(bufs[j], out_hbm.at[wid, h], osems[j])

        def wait_out(j):
            pltpu.make_async_copy(bufs[j], out_hbm.at[wid, 0], osems[j]).wait()

        def scale_row(buf, row):
            qs = [buf[row, pl.ds(q * _L, _L)] for q in range(4)]
            s = qs[0] * qs[0] + qs[1] * qs[1] + qs[2] * qs[2] + qs[3] * qs[3]
            for p in perms:
                s = s + jnp.take_along_axis(s, p, axis=0)
            scale = jnp.where(s > 1.0, _rsqrt16(s), 1.0)
            for q in range(4):
                buf[row, pl.ds(q * _L, _L)] = qs[q] * scale

        def compute(j):
            buf = bufs[j]

            def row_body(r, carry2):
                for u in range(_ROW_UNROLL):
                    scale_row(buf, r * _ROW_UNROLL + u)
                return carry2

            lax.fori_loop(0, _CHUNK // _ROW_UNROLL, row_body, 0)

        def body(h, j, issue_next):
            jj = (j + _NBUF - 1) % _NBUF
            wait_gather(j)
            compute(j)
            wait_out(jj)
            start_out(h, j)
            if issue_next:
                start_gather(h + _NBUF - 1, jj)

        for j in range(_NBUF - 1):
            start_gather(j, j)
        wait_gather(0)
        compute(0)
        start_out(0, 0)
        start_gather(_NBUF - 1, _NBUF - 1)

        def outer(t, carry):
            h0 = 1 + t * _NBUF
            for u in range(_NBUF):
                body(h0 + u, (1 + u) % _NBUF, True)
            return carry

        lax.fori_loop(0, (n_chunks - _NBUF) // _NBUF, outer, 0)
        for h in range(n_chunks - _NBUF + 1, n_chunks):
            body(h, h % _NBUF, False)
        wait_out((n_chunks - 1) % _NBUF)

    return k


def kernel(indices, node_emb):
    bsz, fields = indices.shape
    n_rows = bsz * fields
    assert n_rows % (_NW * _CHUNK) == 0, n_rows
    n_chunks = n_rows // (_NW * _CHUNK)
    idx3 = indices.reshape(_NW, n_chunks, _CHUNK)
    out = _make_kernel(n_chunks)(idx3, node_emb)
    return out.reshape(bsz, fields, node_emb.shape[1])
